# baseline (speedup 1.0000x reference)
import functools

import jax
import jax.numpy as jnp
from jax import lax
from jax.experimental import pallas as pl
from jax.experimental.pallas import tpu as pltpu
from jax.experimental.pallas import tpu_sc as plsc

NC = 2
NS = 16
L = 16
NW = NC * NS
B = 16384
D = 128
BPW = B // NW
C = 32
NCHUNK = BPW // C
NSLOT = 2


def _sc_body(ent_hbm, rel_hbm, hidx_hbm, ridx_hbm, tidx_hbm, out_hbm,
             hidx_v, ridx_v, tidx_v, hbufs, rbufs, tbufs, outbuf, red,
             sem0, sem1):
    wid = lax.axis_index("s") * NC + lax.axis_index("c")
    base_w = wid * BPW
    lane = lax.iota(jnp.int32, L)
    sems = [sem0, sem1]

    sw = pl.ds(base_w, BPW)
    ih = pltpu.async_copy(hidx_hbm.at[sw], hidx_v, sem0)
    ir = pltpu.async_copy(ridx_hbm.at[sw], ridx_v, sem0)
    it = pltpu.async_copy(tidx_hbm.at[sw], tidx_v, sem0)
    ih.wait()
    ir.wait()
    it.wait()

    NSPLIT = max(1, C // 64)
    H = C // NSPLIT

    def fire(c, slot):
        for h in range(NSPLIT):
            s = pl.ds(c * C + h * H, H)
            d = pl.ds(h * H, H)
            pltpu.async_copy(ent_hbm.at[hidx_v.at[s]],
                             hbufs.at[slot].at[d], sems[slot])
            pltpu.async_copy(rel_hbm.at[ridx_v.at[s]],
                             rbufs.at[slot].at[d], sems[slot])
            pltpu.async_copy(ent_hbm.at[tidx_v.at[s]],
                             tbufs.at[slot].at[d], sems[slot])

    def drain(slot):
        for h in range(NSPLIT):
            s = pl.ds(h * H, H)
            pltpu.make_async_copy(ent_hbm.at[hidx_v.at[s]],
                                  hbufs.at[slot].at[s], sems[slot]).wait()
            pltpu.make_async_copy(rel_hbm.at[ridx_v.at[s]],
                                  rbufs.at[slot].at[s], sems[slot]).wait()
            pltpu.make_async_copy(ent_hbm.at[tidx_v.at[s]],
                                  tbufs.at[slot].at[s], sems[slot]).wait()

    def compute(c, slot):
        def group_body(g, gcarry):
            for i in range(L):
                row = g * L + i
                acc = None
                for jv in range(D // L):
                    s = pl.ds(jv * L, L)
                    p = (hbufs[slot, row, s] * rbufs[slot, row, s]
                         * tbufs[slot, row, s])
                    acc = p if acc is None else acc + p
                red[pl.ds(i * L, L)] = acc
            scores = None
            for l in range(L):
                v = plsc.load_gather(red, [lane * L + l])
                scores = v if scores is None else scores + v
            outbuf[pl.ds(c * C + g * L, L)] = scores
            return gcarry

        lax.fori_loop(0, C // L, group_body, 0)

    fire(0, 0)

    def pair_body(p, carry):
        c0 = p * 2
        fire(c0 + 1, 1)
        drain(0)
        compute(c0, 0)

        @pl.when(p + 1 < NCHUNK // 2)
        def _():
            fire(c0 + 2, 0)

        drain(1)
        compute(c0 + 1, 1)
        return carry

    lax.fori_loop(0, NCHUNK // 2, pair_body, 0)

    pltpu.sync_copy(outbuf, out_hbm.at[pl.ds(base_w, BPW)])


_distmult_sc = functools.partial(
    pl.kernel,
    out_type=jax.ShapeDtypeStruct((B,), jnp.float32),
    mesh=plsc.VectorSubcoreMesh(
        core_axis_name="c", subcore_axis_name="s",
        num_cores=NC, num_subcores=NS),
    scratch_types=[
        pltpu.VMEM((BPW,), jnp.int32),
        pltpu.VMEM((BPW,), jnp.int32),
        pltpu.VMEM((BPW,), jnp.int32),
        pltpu.VMEM((NSLOT, C, D), jnp.float32),
        pltpu.VMEM((NSLOT, C, D), jnp.float32),
        pltpu.VMEM((NSLOT, C, D), jnp.float32),
        pltpu.VMEM((BPW,), jnp.float32),
        pltpu.VMEM((L * L,), jnp.float32),
        pltpu.SemaphoreType.DMA,
        pltpu.SemaphoreType.DMA,
    ],
    compiler_params=pltpu.CompilerParams(needs_layout_passes=False),
)(_sc_body)


@jax.jit
def kernel(entity_emb, relation_emb, head_index, relation_index, tail_index):
    return _distmult_sc(
        entity_emb,
        relation_emb,
        head_index.astype(jnp.int32),
        relation_index.astype(jnp.int32),
        tail_index.astype(jnp.int32),
    )


# speedup vs baseline: 1.0449x; 1.0449x over previous
import functools

import jax
import jax.numpy as jnp
from jax import lax
from jax.experimental import pallas as pl
from jax.experimental.pallas import tpu as pltpu
from jax.experimental.pallas import tpu_sc as plsc

NC = 2
NS = 16
L = 16
NW = NC * NS
B = 16384
D = 128
BPW = B // NW
C = 64
NCHUNK = BPW // C
NSLOT = 2


def _sc_body(ent_hbm, rel_hbm, hidx_hbm, ridx_hbm, tidx_hbm, out_hbm,
             hidx_v, ridx_v, tidx_v, hbufs, rbufs, tbufs, outbuf, red,
             sem0, sem1):
    wid = lax.axis_index("s") * NC + lax.axis_index("c")
    base_w = wid * BPW
    lane = lax.iota(jnp.int32, L)
    sems = [sem0, sem1]

    sw = pl.ds(base_w, BPW)
    ih = pltpu.async_copy(hidx_hbm.at[sw], hidx_v, sem0)
    ir = pltpu.async_copy(ridx_hbm.at[sw], ridx_v, sem0)
    it = pltpu.async_copy(tidx_hbm.at[sw], tidx_v, sem0)
    ih.wait()
    ir.wait()
    it.wait()

    NSPLIT = max(1, C // 64)
    H = C // NSPLIT

    def fire(c, slot):
        for h in range(NSPLIT):
            s = pl.ds(c * C + h * H, H)
            d = pl.ds(h * H, H)
            pltpu.async_copy(ent_hbm.at[hidx_v.at[s]],
                             hbufs.at[slot].at[d], sems[slot])
            pltpu.async_copy(rel_hbm.at[ridx_v.at[s]],
                             rbufs.at[slot].at[d], sems[slot])
            pltpu.async_copy(ent_hbm.at[tidx_v.at[s]],
                             tbufs.at[slot].at[d], sems[slot])

    def drain(slot):
        for h in range(NSPLIT):
            s = pl.ds(h * H, H)
            pltpu.make_async_copy(ent_hbm.at[hidx_v.at[s]],
                                  hbufs.at[slot].at[s], sems[slot]).wait()
            pltpu.make_async_copy(rel_hbm.at[ridx_v.at[s]],
                                  rbufs.at[slot].at[s], sems[slot]).wait()
            pltpu.make_async_copy(ent_hbm.at[tidx_v.at[s]],
                                  tbufs.at[slot].at[s], sems[slot]).wait()

    def compute(c, slot):
        def group_body(g, gcarry):
            for i in range(L):
                row = g * L + i
                acc = None
                for jv in range(D // L):
                    s = pl.ds(jv * L, L)
                    p = (hbufs[slot, row, s] * rbufs[slot, row, s]
                         * tbufs[slot, row, s])
                    acc = p if acc is None else acc + p
                red[pl.ds(i * L, L)] = acc
            scores = None
            for l in range(L):
                v = plsc.load_gather(red, [lane * L + l])
                scores = v if scores is None else scores + v
            outbuf[pl.ds(c * C + g * L, L)] = scores
            return gcarry

        lax.fori_loop(0, C // L, group_body, 0)

    fire(0, 0)

    def pair_body(p, carry):
        c0 = p * 2
        fire(c0 + 1, 1)
        drain(0)
        compute(c0, 0)

        @pl.when(p + 1 < NCHUNK // 2)
        def _():
            fire(c0 + 2, 0)

        drain(1)
        compute(c0 + 1, 1)
        return carry

    lax.fori_loop(0, NCHUNK // 2, pair_body, 0)

    pltpu.sync_copy(outbuf, out_hbm.at[pl.ds(base_w, BPW)])


_distmult_sc = functools.partial(
    pl.kernel,
    out_type=jax.ShapeDtypeStruct((B,), jnp.float32),
    mesh=plsc.VectorSubcoreMesh(
        core_axis_name="c", subcore_axis_name="s",
        num_cores=NC, num_subcores=NS),
    scratch_types=[
        pltpu.VMEM((BPW,), jnp.int32),
        pltpu.VMEM((BPW,), jnp.int32),
        pltpu.VMEM((BPW,), jnp.int32),
        pltpu.VMEM((NSLOT, C, D), jnp.float32),
        pltpu.VMEM((NSLOT, C, D), jnp.float32),
        pltpu.VMEM((NSLOT, C, D), jnp.float32),
        pltpu.VMEM((BPW,), jnp.float32),
        pltpu.VMEM((L * L,), jnp.float32),
        pltpu.SemaphoreType.DMA,
        pltpu.SemaphoreType.DMA,
    ],
    compiler_params=pltpu.CompilerParams(needs_layout_passes=False),
)(_sc_body)


@jax.jit
def kernel(entity_emb, relation_emb, head_index, relation_index, tail_index):
    return _distmult_sc(
        entity_emb,
        relation_emb,
        head_index.astype(jnp.int32),
        relation_index.astype(jnp.int32),
        tail_index.astype(jnp.int32),
    )
